# baseline (speedup 1.0000x reference)
import jax
import jax.numpy as jnp
from jax.experimental import pallas as pl
from jax.experimental.pallas import tpu as pltpu

K_STATIC = 64


def _prep_kernel(a_ref, b_ref, an_ref, bc_ref):
    a = a_ref[...]
    s = jnp.sum(a * a, axis=0, keepdims=True)
    an_ref[...] = (a * (1.0 / jnp.sqrt(s))).astype(jnp.bfloat16)
    bc_ref[...] = b_ref[...].astype(jnp.bfloat16)


def _cast_kernel(x_ref, xc_ref):
    xc_ref[...] = x_ref[...].astype(jnp.bfloat16)


def _mm1_kernel(x_ref, an_ref, out_ref):
    out_ref[...] = jnp.dot(x_ref[...], an_ref[...],
                           preferred_element_type=jnp.float32)


def _select_kernel(y_ref, out_ref, outb_ref):
    y = y_ref[...]
    bits = jax.lax.bitcast_convert_type(jnp.abs(y), jnp.int32)
    hi0 = jnp.max(bits, axis=1, keepdims=True) + 1
    lo0 = jnp.zeros_like(hi0)
    cnt0 = jnp.full_like(hi0, y.shape[1], dtype=jnp.float32)
    kf = float(K_STATIC)

    def step(lo, hi, cntlo):
        mid = lo + ((hi - lo) >> 1)
        cnt = jnp.sum(jnp.where(bits >= mid, 1.0, 0.0),
                      axis=1, keepdims=True)
        ge = cnt >= kf
        return (jnp.where(ge, mid, lo),
                jnp.where(ge, hi, mid),
                jnp.where(ge, cnt, cntlo))

    def cond(carry):
        t, _, _, cntlo = carry
        notdone = jnp.sum(jnp.where(cntlo == kf, 0.0, 1.0))
        return jnp.logical_and(t < 8, notdone > 0.0)

    def body(carry):
        t, lo, hi, cntlo = carry
        lo, hi, cntlo = step(lo, hi, cntlo)
        lo, hi, cntlo = step(lo, hi, cntlo)
        lo, hi, cntlo = step(lo, hi, cntlo)
        lo, hi, cntlo = step(lo, hi, cntlo)
        return (t + 1, lo, hi, cntlo)

    _, lo, _, _ = jax.lax.while_loop(cond, body, (0, lo0, hi0, cnt0))
    masked = jnp.where(bits >= lo, y, 0.0)
    out_ref[...] = masked
    outb_ref[...] = masked.astype(jnp.bfloat16)


def _mm2_kernel(m_ref, b_ref, out_ref):
    out_ref[...] = jnp.dot(m_ref[...], b_ref[...],
                           preferred_element_type=jnp.float32)


def kernel(x, A, B, topk):
    del topk
    M, F = x.shape
    N = A.shape[1]
    G = B.shape[1]

    bn_p = min(512, N)
    An, Bc = pl.pallas_call(
        _prep_kernel,
        grid=(N // bn_p,),
        in_specs=[pl.BlockSpec((F, bn_p), lambda j: (0, j)),
                  pl.BlockSpec((N, bn_p), lambda j: (0, j))],
        out_specs=[pl.BlockSpec((F, bn_p), lambda j: (0, j)),
                   pl.BlockSpec((N, bn_p), lambda j: (0, j))],
        out_shape=[jax.ShapeDtypeStruct((F, N), jnp.bfloat16),
                   jax.ShapeDtypeStruct((N, G), jnp.bfloat16)],
    )(A, B)

    bm_c = min(512, M)
    xc = pl.pallas_call(
        _cast_kernel,
        grid=(M // bm_c,),
        in_specs=[pl.BlockSpec((bm_c, F), lambda i: (i, 0))],
        out_specs=pl.BlockSpec((bm_c, F), lambda i: (i, 0)),
        out_shape=jax.ShapeDtypeStruct((M, F), jnp.bfloat16),
    )(x)

    bm1 = min(512, M)
    bn1 = min(2048, N)
    y_raw = pl.pallas_call(
        _mm1_kernel,
        grid=(N // bn1, M // bm1),
        in_specs=[
            pl.BlockSpec((bm1, F), lambda n, i: (i, 0)),
            pl.BlockSpec((F, bn1), lambda n, i: (0, n)),
        ],
        out_specs=pl.BlockSpec((bm1, bn1), lambda n, i: (i, n)),
        out_shape=jax.ShapeDtypeStruct((M, N), jnp.float32),
        compiler_params=pltpu.CompilerParams(
            dimension_semantics=("arbitrary", "arbitrary")),
    )(xc, An)

    bm_s = min(512, M)
    inner, inner_bf = pl.pallas_call(
        _select_kernel,
        grid=(M // bm_s,),
        in_specs=[pl.BlockSpec((bm_s, N), lambda i: (i, 0))],
        out_specs=[pl.BlockSpec((bm_s, N), lambda i: (i, 0)),
                   pl.BlockSpec((bm_s, N), lambda i: (i, 0))],
        out_shape=[jax.ShapeDtypeStruct((M, N), jnp.float32),
                   jax.ShapeDtypeStruct((M, N), jnp.bfloat16)],
        compiler_params=pltpu.CompilerParams(
            dimension_semantics=("arbitrary",)),
    )(y_raw)

    bm2 = min(512, M)
    bg2 = min(2048, G)
    out = pl.pallas_call(
        _mm2_kernel,
        grid=(G // bg2, M // bm2),
        in_specs=[
            pl.BlockSpec((bm2, N), lambda g, i: (i, 0)),
            pl.BlockSpec((N, bg2), lambda g, i: (0, g)),
        ],
        out_specs=pl.BlockSpec((bm2, bg2), lambda g, i: (i, g)),
        out_shape=jax.ShapeDtypeStruct((M, G), jnp.float32),
        compiler_params=pltpu.CompilerParams(
            dimension_semantics=("arbitrary", "arbitrary")),
    )(inner_bf, Bc)

    return out, inner


# speedup vs baseline: 1.0047x; 1.0047x over previous
import jax
import jax.numpy as jnp
from jax.experimental import pallas as pl
from jax.experimental.pallas import tpu as pltpu

K_STATIC = 64


def _prep_kernel(a_ref, b_ref, an_ref, bc_ref):
    a = a_ref[...]
    s = jnp.sum(a * a, axis=0, keepdims=True)
    an_ref[...] = (a * (1.0 / jnp.sqrt(s))).astype(jnp.bfloat16)
    bc_ref[...] = b_ref[...].astype(jnp.bfloat16)


def _cast_kernel(x_ref, xc_ref):
    xc_ref[...] = x_ref[...].astype(jnp.bfloat16)


def _mm1_kernel(x_ref, an_ref, out_ref):
    out_ref[...] = jnp.dot(x_ref[...], an_ref[...],
                           preferred_element_type=jnp.float32)


def _select_kernel(y_ref, out_ref):
    y = y_ref[...]
    bits = jax.lax.bitcast_convert_type(jnp.abs(y), jnp.int32)
    hi0 = jnp.max(bits, axis=1, keepdims=True) + 1
    lo0 = jnp.zeros_like(hi0)
    cnt0 = jnp.full_like(hi0, y.shape[1], dtype=jnp.float32)
    kf = float(K_STATIC)

    def step(lo, hi, cntlo):
        mid = lo + ((hi - lo) >> 1)
        cnt = jnp.sum(jnp.where(bits >= mid, 1.0, 0.0),
                      axis=1, keepdims=True)
        ge = cnt >= kf
        return (jnp.where(ge, mid, lo),
                jnp.where(ge, hi, mid),
                jnp.where(ge, cnt, cntlo))

    def cond(carry):
        t, _, _, cntlo = carry
        notdone = jnp.sum(jnp.where(cntlo == kf, 0.0, 1.0))
        return jnp.logical_and(t < 8, notdone > 0.0)

    def body(carry):
        t, lo, hi, cntlo = carry
        lo, hi, cntlo = step(lo, hi, cntlo)
        lo, hi, cntlo = step(lo, hi, cntlo)
        lo, hi, cntlo = step(lo, hi, cntlo)
        lo, hi, cntlo = step(lo, hi, cntlo)
        return (t + 1, lo, hi, cntlo)

    _, lo, _, _ = jax.lax.while_loop(cond, body, (0, lo0, hi0, cnt0))
    out_ref[...] = jnp.where(bits >= lo, y, 0.0)


def _mm2_kernel(m_ref, b_ref, out_ref):
    out_ref[...] = jnp.dot(m_ref[...].astype(jnp.bfloat16), b_ref[...],
                           preferred_element_type=jnp.float32)


def kernel(x, A, B, topk):
    del topk
    M, F = x.shape
    N = A.shape[1]
    G = B.shape[1]

    bn_p = min(512, N)
    An, Bc = pl.pallas_call(
        _prep_kernel,
        grid=(N // bn_p,),
        in_specs=[pl.BlockSpec((F, bn_p), lambda j: (0, j)),
                  pl.BlockSpec((N, bn_p), lambda j: (0, j))],
        out_specs=[pl.BlockSpec((F, bn_p), lambda j: (0, j)),
                   pl.BlockSpec((N, bn_p), lambda j: (0, j))],
        out_shape=[jax.ShapeDtypeStruct((F, N), jnp.bfloat16),
                   jax.ShapeDtypeStruct((N, G), jnp.bfloat16)],
    )(A, B)

    bm_c = min(512, M)
    xc = pl.pallas_call(
        _cast_kernel,
        grid=(M // bm_c,),
        in_specs=[pl.BlockSpec((bm_c, F), lambda i: (i, 0))],
        out_specs=pl.BlockSpec((bm_c, F), lambda i: (i, 0)),
        out_shape=jax.ShapeDtypeStruct((M, F), jnp.bfloat16),
    )(x)

    bm1 = min(512, M)
    bn1 = min(2048, N)
    y_raw = pl.pallas_call(
        _mm1_kernel,
        grid=(N // bn1, M // bm1),
        in_specs=[
            pl.BlockSpec((bm1, F), lambda n, i: (i, 0)),
            pl.BlockSpec((F, bn1), lambda n, i: (0, n)),
        ],
        out_specs=pl.BlockSpec((bm1, bn1), lambda n, i: (i, n)),
        out_shape=jax.ShapeDtypeStruct((M, N), jnp.float32),
        compiler_params=pltpu.CompilerParams(
            dimension_semantics=("arbitrary", "arbitrary")),
    )(xc, An)

    bm_s = min(512, M)
    inner = pl.pallas_call(
        _select_kernel,
        grid=(M // bm_s,),
        in_specs=[pl.BlockSpec((bm_s, N), lambda i: (i, 0))],
        out_specs=pl.BlockSpec((bm_s, N), lambda i: (i, 0)),
        out_shape=jax.ShapeDtypeStruct((M, N), jnp.float32),
        compiler_params=pltpu.CompilerParams(
            dimension_semantics=("arbitrary",)),
    )(y_raw)

    bm2 = min(256, M)
    out = pl.pallas_call(
        _mm2_kernel,
        grid=(M // bm2,),
        in_specs=[
            pl.BlockSpec((bm2, N), lambda i: (i, 0)),
            pl.BlockSpec((N, G), lambda i: (0, 0)),
        ],
        out_specs=pl.BlockSpec((bm2, G), lambda i: (i, 0)),
        out_shape=jax.ShapeDtypeStruct((M, G), jnp.float32),
        compiler_params=pltpu.CompilerParams(
            dimension_semantics=("arbitrary",)),
    )(inner, Bc)

    return out, inner
